# baseline (speedup 1.0000x reference)
import jax
import jax.numpy as jnp
from jax import lax
from jax.experimental import pallas as pl
from jax.experimental.pallas import tpu as pltpu
from jax.experimental.pallas import tpu_sc as plsc

_Q_LEN = 2048
_OUT_DIM = 12
_NC = 2
_NS = 16
_GK_PAD = 4160


def _body(emb_hbm, out_hbm, emb_ts, gk, bidx, mbuf, mr, sem):
    c = lax.axis_index("c")
    s = lax.axis_index("s")
    lanes = lax.iota(jnp.int32, 16)

    pltpu.sync_copy(emb_hbm, emb_ts)

    def fill_bidx(jd, carry):
        q = 16 * jd + lanes
        n = jnp.abs(q - (_Q_LEN - 1))
        side = jnp.where(q > (_Q_LEN - 1), 16, 0)
        nsq_f = (n * n).astype(jnp.float32)
        e = lax.bitcast_convert_type(nsq_f, jnp.int32) >> 23
        val_large = jnp.minimum(e - 125, 15)
        bucket = side + jnp.where(n < 8, n, val_large)
        bidx[pl.ds(16 * jd, 16)] = bucket * _OUT_DIM
        return carry

    lax.fori_loop(0, 4096 // 16, fill_bidx, 0)

    def drain(t, carry):
        pltpu.make_async_copy(
            mr.at[0, pl.ds(0, 16), pl.ds(0, 8), :],
            out_hbm.at[0, 0],
            sem,
        ).wait()
        return carry

    for r in range(_OUT_DIM):
        k = r
        slot = r % 3
        if r >= 2:
            lax.fori_loop(0, 8, drain, 0)
        plsc.subcore_barrier()

        def fill_gk(jd, carry, _k=k):
            for u in range(8):
                off = 16 * (8 * jd + u)
                bv = bidx[pl.ds(off, 16)]
                gk[pl.ds(off, 16)] = plsc.load_gather(emb_ts, [bv + _k])
            return carry

        lax.fori_loop(0, 4096 // 128, fill_gk, 0)

        row0 = 128 * s

        def br(r2, carry):
            for v in range(2):
                rr = 2 * r2 + v
                q = row0 + rr
                base = 16 * (q >> 3) - 113 - 8 * c - (q & 7)
                for cc in range(8):
                    idx = jnp.maximum((base + 16 * cc) + lanes, 0)
                    mbuf[rr >> 6, rr & 63, pl.ds(16 * cc, 16)] = (
                        plsc.load_gather(gk, [idx])
                    )
            return carry

        lax.fori_loop(0, 64, br, 0)
        pltpu.sync_copy(
            mbuf.at[:, :, :],
            mr.at[slot, pl.ds(2 * s, 2), :, :],
        )
        plsc.subcore_barrier()

        def fire(f, carry, _k=k, _slot=slot):
            u = 8 * s + f
            ti = 2 * u + c
            r0 = 8 * (135 - u)
            m0 = r0 >> 6
            t0 = pl.multiple_of(r0 & 63, 8)
            pltpu.async_copy(
                mr.at[_slot, pl.ds(m0, 16), pl.ds(t0, 8), :],
                out_hbm.at[_k, ti],
                sem,
            )
            return carry

        lax.fori_loop(0, 8, fire, 0)

    lax.fori_loop(0, 16, drain, 0)


_sc_expand = pl.kernel(
    _body,
    out_type=jax.ShapeDtypeStruct((_OUT_DIM, 256, 16, 8, 128), jnp.float32),
    mesh=plsc.VectorSubcoreMesh(
        core_axis_name="c", subcore_axis_name="s", num_cores=_NC, num_subcores=_NS
    ),
    scratch_types=[
        pltpu.VMEM((384,), jnp.float32),
        pltpu.VMEM((_GK_PAD,), jnp.float32),
        pltpu.VMEM((_GK_PAD,), jnp.int32),
        pltpu.VMEM((2, 64, 128), jnp.float32),
        pltpu.VMEM_SHARED((3, 32, 64, 128), jnp.float32),
        pltpu.SemaphoreType.DMA,
    ],
    compiler_params=pltpu.CompilerParams(
        needs_layout_passes=False, use_tc_tiling_on_sc=True
    ),
)


@jax.jit
def kernel(q, v, embeddings):
    del q, v
    out = _sc_expand(embeddings.reshape(-1))
    return jnp.transpose(out, (1, 3, 2, 4, 0)).reshape(_Q_LEN, _Q_LEN, _OUT_DIM)
